# baseline (speedup 1.0000x reference)
import jax
import jax.numpy as jnp
from jax import lax
from jax.experimental import pallas as pl
from jax.experimental.pallas import tpu as pltpu
from jax.experimental.pallas import tpu_sc as plsc

N = 4_000_000
BINS = 5
CHUNK = 16_000
NCHUNKS = N // CHUNK
TCC = 168
SCC = NCHUNKS - TCC
TC_RB = 200
TC_STEPS = TCC * 125 // TC_RB
NC, NS = 2, 16
NW = NC * NS
VPC = CHUNK // 16

_BASE = SCC // NW
_EXTRA = SCC % NW

_LT = (
    0.2876902085936771, -1.333342676597351, 0.8865566226237033,
    -0.7874456068595445, 0.8869645527936711, -0.9538804877894336,
)


BPC = CHUNK // 128


def _sc_body(x_hbm, y_hbm, out_hbm,
             xb0, yb0, xb1, yb1, acc_s, acc_c, sx0, sy0, sx1, sy1):
    cid = lax.axis_index("c")
    sid = lax.axis_index("s")
    wid = sid * NC + cid
    nch = jnp.where(wid < _EXTRA, _BASE + 1, _BASE)
    start = TCC + wid * _BASE + jnp.minimum(wid, _EXTRA)

    zeros16 = jnp.zeros((16,), jnp.float32)
    for k in range(BINS):
        acc_s[pl.ds(k * 16, 16)] = zeros16
        acc_c[pl.ds(k * 16, 16)] = zeros16

    iota = lax.iota(jnp.int32, 16)
    ones16 = jnp.ones((16,), jnp.float32)

    def start_dma(c, xb, yb, sx, sy):
        pltpu.async_copy(x_hbm.at[pl.ds(c * BPC, BPC), 1, :], xb, sx)
        pltpu.async_copy(y_hbm.at[pl.ds(c * CHUNK, CHUNK)], yb, sy)

    def wait_dma(c, xb, yb, sx, sy):
        pltpu.make_async_copy(x_hbm.at[pl.ds(c * BPC, BPC), 1, :], xb, sx).wait()
        pltpu.make_async_copy(y_hbm.at[pl.ds(c * CHUNK, CHUNK)], yb, sy).wait()

    def process(xb, yb):
        @plsc.parallel_loop(0, BPC, unroll=1)
        def _(blk):
            for k in range(8):
                y = yb[pl.ds(blk * 128 + k * 16, 16)]
                x = xb[blk, pl.ds(k * 16, 16)]
                e = jnp.exp(-x)
                r = 1.0 / (1.0 + e)
                g = jnp.abs(r - y)
                u = jnp.maximum(r, 1.0 - r) - 0.75
                p = jnp.float32(_LT[5])
                for j in (4, 3, 2, 1, 0):
                    p = p * u + _LT[j]
                pe = jnp.maximum(x, 0.0) - x * y + p
                b = jnp.minimum((g * 5.0).astype(jnp.int32), 4)
                idx = b * 16 + iota
                plsc.addupdate_scatter(acc_s, [idx], pe)
                plsc.addupdate_scatter(acc_c, [idx], ones16)

    start_dma(start, xb0, yb0, sx0, sy0)

    def chunk_body(j, carry):
        c = start + j
        even = (j % 2) == 0

        @pl.when(even)
        def _():
            @pl.when(j + 1 < nch)
            def _():
                start_dma(c + 1, xb1, yb1, sx1, sy1)
            wait_dma(c, xb0, yb0, sx0, sy0)
            process(xb0, yb0)

        @pl.when(jnp.logical_not(even))
        def _():
            @pl.when(j + 1 < nch)
            def _():
                start_dma(c + 1, xb0, yb0, sx0, sy0)
            wait_dma(c, xb1, yb1, sx1, sy1)
            process(xb1, yb1)

        return carry

    lax.fori_loop(0, nch, chunk_body, 0)
    pltpu.sync_copy(acc_s, out_hbm.at[pl.ds(wid * 160, 80)])
    pltpu.sync_copy(acc_c, out_hbm.at[pl.ds(wid * 160 + 80, 80)])


@jax.jit
def _ghm_sc(x, targets):
    mesh = plsc.VectorSubcoreMesh(
        core_axis_name="c", subcore_axis_name="s", num_cores=NC, num_subcores=NS
    )
    run = pl.kernel(
        _sc_body,
        out_type=jax.ShapeDtypeStruct((NW * 160,), jnp.float32),
        mesh=mesh,
        scratch_types=[
            pltpu.VMEM((BPC, 128), jnp.float32),
            pltpu.VMEM((CHUNK,), jnp.float32),
            pltpu.VMEM((BPC, 128), jnp.float32),
            pltpu.VMEM((CHUNK,), jnp.float32),
            pltpu.VMEM((80,), jnp.float32),
            pltpu.VMEM((80,), jnp.float32),
            pltpu.SemaphoreType.DMA,
            pltpu.SemaphoreType.DMA,
            pltpu.SemaphoreType.DMA,
            pltpu.SemaphoreType.DMA,
        ],
        compiler_params=pltpu.CompilerParams(needs_layout_passes=False),
    )
    return run(x, targets)


RTC = TCC * 125


def _tc_body(x3_ref, y_ref, out_ref, xb, yb):
    pltpu.sync_copy(x3_ref.at[pl.ds(0, RTC), 1, :], xb)
    pltpu.sync_copy(y_ref.at[pl.ds(0, RTC), :], yb)

    def strip(s, acc):
        x = xb[pl.ds(8 * s, 8), :]
        y = yb[pl.ds(8 * s, 8), :]
        e = jnp.exp(-x)
        r = 1.0 / (1.0 + e)
        g = jnp.abs(r - y)
        pe = jnp.maximum(x, 0.0) - x * y - jnp.log(jnp.maximum(r, 1.0 - r))
        acc0 = acc[0] + pe
        new = [acc0]
        for k in (1, 2, 3, 4):
            m = g >= (k / 5.0)
            new.append(acc[k] + jnp.where(m, pe, 0.0))
        for k in (1, 2, 3, 4):
            m = g >= (k / 5.0)
            new.append(acc[4 + k] + jnp.where(m, 1.0, 0.0))
        return tuple(new)

    z8 = jnp.zeros((8, 128), jnp.float32)
    acc = lax.fori_loop(0, RTC // 8, strip, (z8,) * 9, unroll=4)
    for k in range(9):
        out_ref[8 * k:8 * k + 8, :] = acc[k]


@jax.jit
def _ghm_tc(x3, y2):
    return pl.pallas_call(
        _tc_body,
        in_specs=[
            pl.BlockSpec(memory_space=pl.ANY),
            pl.BlockSpec(memory_space=pl.ANY),
        ],
        out_specs=pl.BlockSpec(memory_space=pltpu.MemorySpace.VMEM),
        out_shape=jax.ShapeDtypeStruct((72, 128), jnp.float32),
        scratch_shapes=[
            pltpu.VMEM((RTC, 128), jnp.float32),
            pltpu.VMEM((RTC, 128), jnp.float32),
        ],
    )(x3, y2)


def kernel(logits, targets):
    x3 = logits.reshape(N // 128, 128, 2).transpose(0, 2, 1)
    part = _ghm_sc(x3, targets)
    y2 = targets.reshape(N // 128, 128)
    tcp = _ghm_tc(x3, y2)
    p = part.reshape(NW, 2, BINS, 16)
    s_b = jnp.sum(p[:, 0], axis=(0, 2))
    c_b = jnp.sum(p[:, 1], axis=(0, 2))
    q = jnp.sum(tcp.reshape(9, 8, 128), axis=(1, 2))
    s_t, t1, t2, t3, t4, c1, c2, c3, c4 = (q[i] for i in range(9))
    ntc = float(TCC * CHUNK)
    s_b = s_b + jnp.stack([s_t - t1, t1 - t2, t2 - t3, t3 - t4, t4])
    c_b = c_b + jnp.stack([ntc - c1, c1 - c2, c2 - c3, c3 - c4, c4])
    total = float(logits.size)
    w_b = jnp.where(c_b > 0, total / ((1.0 - 0.5) * c_b), 0.0)
    return jnp.sum(w_b * s_b) / targets.shape[0]


# speedup vs baseline: 1.1109x; 1.1109x over previous
import jax
import jax.numpy as jnp
from jax import lax
from jax.experimental import pallas as pl
from jax.experimental.pallas import tpu as pltpu
from jax.experimental.pallas import tpu_sc as plsc

N = 4_000_000
BINS = 5
CHUNK = 16_000
NCHUNKS = N // CHUNK
TCC = 160
SCC = NCHUNKS - TCC
TC_RB = 200
TC_STEPS = TCC * 125 // TC_RB
NC, NS = 2, 16
NW = NC * NS
VPC = CHUNK // 16

_BASE = SCC // NW
_EXTRA = SCC % NW

_LT = (
    0.2876902085936771, -1.333342676597351, 0.8865566226237033,
    -0.7874456068595445, 0.8869645527936711, -0.9538804877894336,
)


BPC = CHUNK // 128


def _sc_body(x_hbm, y_hbm, out_hbm,
             xb0, yb0, xb1, yb1, acc_s, acc_c, sx0, sy0, sx1, sy1):
    cid = lax.axis_index("c")
    sid = lax.axis_index("s")
    wid = sid * NC + cid
    nch = jnp.where(wid < _EXTRA, _BASE + 1, _BASE)
    start = TCC + wid * _BASE + jnp.minimum(wid, _EXTRA)

    zeros16 = jnp.zeros((16,), jnp.float32)
    for k in range(BINS):
        acc_s[pl.ds(k * 16, 16)] = zeros16
        acc_c[pl.ds(k * 16, 16)] = zeros16

    iota = lax.iota(jnp.int32, 16)
    ones16 = jnp.ones((16,), jnp.float32)

    def start_dma(c, xb, yb, sx, sy):
        pltpu.async_copy(x_hbm.at[pl.ds(c * BPC, BPC), 1, :], xb, sx)
        pltpu.async_copy(y_hbm.at[pl.ds(c * CHUNK, CHUNK)], yb, sy)

    def wait_dma(c, xb, yb, sx, sy):
        pltpu.make_async_copy(x_hbm.at[pl.ds(c * BPC, BPC), 1, :], xb, sx).wait()
        pltpu.make_async_copy(y_hbm.at[pl.ds(c * CHUNK, CHUNK)], yb, sy).wait()

    def process(xb, yb):
        @plsc.parallel_loop(0, BPC, unroll=1)
        def _(blk):
            for k in range(8):
                y = yb[pl.ds(blk * 128 + k * 16, 16)]
                x = xb[blk, pl.ds(k * 16, 16)]
                e = jnp.exp(-x)
                r = 1.0 / (1.0 + e)
                g = jnp.abs(r - y)
                u = jnp.maximum(r, 1.0 - r) - 0.75
                p = jnp.float32(_LT[5])
                for j in (4, 3, 2, 1, 0):
                    p = p * u + _LT[j]
                pe = jnp.maximum(x, 0.0) - x * y + p
                b = jnp.minimum((g * 5.0).astype(jnp.int32), 4)
                idx = b * 16 + iota
                plsc.addupdate_scatter(acc_s, [idx], pe)
                plsc.addupdate_scatter(acc_c, [idx], ones16)

    start_dma(start, xb0, yb0, sx0, sy0)

    def chunk_body(j, carry):
        c = start + j
        even = (j % 2) == 0

        @pl.when(even)
        def _():
            @pl.when(j + 1 < nch)
            def _():
                start_dma(c + 1, xb1, yb1, sx1, sy1)
            wait_dma(c, xb0, yb0, sx0, sy0)
            process(xb0, yb0)

        @pl.when(jnp.logical_not(even))
        def _():
            @pl.when(j + 1 < nch)
            def _():
                start_dma(c + 1, xb0, yb0, sx0, sy0)
            wait_dma(c, xb1, yb1, sx1, sy1)
            process(xb1, yb1)

        return carry

    lax.fori_loop(0, nch, chunk_body, 0)
    pltpu.sync_copy(acc_s, out_hbm.at[pl.ds(wid * 160, 80)])
    pltpu.sync_copy(acc_c, out_hbm.at[pl.ds(wid * 160 + 80, 80)])


@jax.jit
def _ghm_sc(x, targets):
    mesh = plsc.VectorSubcoreMesh(
        core_axis_name="c", subcore_axis_name="s", num_cores=NC, num_subcores=NS
    )
    run = pl.kernel(
        _sc_body,
        out_type=jax.ShapeDtypeStruct((NW * 160,), jnp.float32),
        mesh=mesh,
        scratch_types=[
            pltpu.VMEM((BPC, 128), jnp.float32),
            pltpu.VMEM((CHUNK,), jnp.float32),
            pltpu.VMEM((BPC, 128), jnp.float32),
            pltpu.VMEM((CHUNK,), jnp.float32),
            pltpu.VMEM((80,), jnp.float32),
            pltpu.VMEM((80,), jnp.float32),
            pltpu.SemaphoreType.DMA,
            pltpu.SemaphoreType.DMA,
            pltpu.SemaphoreType.DMA,
            pltpu.SemaphoreType.DMA,
        ],
        compiler_params=pltpu.CompilerParams(needs_layout_passes=False),
    )
    return run(x, targets)


RTC = TCC * 125
TC_PH = 4
HR = RTC // TC_PH


def _tc_body(x3_ref, y_ref, out_ref, xb, yb, sx, sy):
    def start(h):
        s = h % 2
        pltpu.make_async_copy(
            x3_ref.at[pl.ds(h * HR, HR), 1, :], xb.at[s], sx.at[s]).start()
        pltpu.make_async_copy(
            y_ref.at[pl.ds(h * HR, HR), :], yb.at[s], sy.at[s]).start()

    def wait(h):
        s = h % 2
        pltpu.make_async_copy(
            x3_ref.at[pl.ds(h * HR, HR), 1, :], xb.at[s], sx.at[s]).wait()
        pltpu.make_async_copy(
            y_ref.at[pl.ds(h * HR, HR), :], yb.at[s], sy.at[s]).wait()

    def strip_fn(xh, yh):
        def strip(s, acc):
            x = xh[pl.ds(8 * s, 8), :]
            y = yh[pl.ds(8 * s, 8), :]
            e = jnp.exp(-x)
            r = 1.0 / (1.0 + e)
            g = jnp.abs(r - y)
            pe = jnp.maximum(x, 0.0) - x * y - jnp.log(jnp.maximum(r, 1.0 - r))
            new = [acc[0] + pe]
            for k in (1, 2, 3, 4):
                m = g >= (k / 5.0)
                new.append(acc[k] + jnp.where(m, pe, 0.0))
            for k in (1, 2, 3, 4):
                m = g >= (k / 5.0)
                new.append(acc[4 + k] + jnp.where(m, 1.0, 0.0))
            return tuple(new)
        return strip

    z8 = jnp.zeros((8, 128), jnp.float32)
    acc = (z8,) * 9
    start(0)
    for h in range(TC_PH):
        if h + 1 < TC_PH:
            start(h + 1)
        wait(h)
        s = h % 2
        acc = lax.fori_loop(0, HR // 8, strip_fn(xb.at[s], yb.at[s]), acc,
                            unroll=4)
    for k in range(9):
        out_ref[8 * k:8 * k + 8, :] = acc[k]


@jax.jit
def _ghm_tc(x3, y2):
    return pl.pallas_call(
        _tc_body,
        in_specs=[
            pl.BlockSpec(memory_space=pl.ANY),
            pl.BlockSpec(memory_space=pl.ANY),
        ],
        out_specs=pl.BlockSpec(memory_space=pltpu.MemorySpace.VMEM),
        out_shape=jax.ShapeDtypeStruct((72, 128), jnp.float32),
        scratch_shapes=[
            pltpu.VMEM((2, HR, 128), jnp.float32),
            pltpu.VMEM((2, HR, 128), jnp.float32),
            pltpu.SemaphoreType.DMA((2,)),
            pltpu.SemaphoreType.DMA((2,)),
        ],
    )(x3, y2)


def kernel(logits, targets):
    x3 = logits.reshape(N // 128, 128, 2).transpose(0, 2, 1)
    part = _ghm_sc(x3, targets)
    y2 = targets.reshape(N // 128, 128)
    tcp = _ghm_tc(x3, y2)
    p = part.reshape(NW, 2, BINS, 16)
    s_b = jnp.sum(p[:, 0], axis=(0, 2))
    c_b = jnp.sum(p[:, 1], axis=(0, 2))
    q = jnp.sum(tcp.reshape(9, 8, 128), axis=(1, 2))
    s_t, t1, t2, t3, t4, c1, c2, c3, c4 = (q[i] for i in range(9))
    ntc = float(TCC * CHUNK)
    s_b = s_b + jnp.stack([s_t - t1, t1 - t2, t2 - t3, t3 - t4, t4])
    c_b = c_b + jnp.stack([ntc - c1, c1 - c2, c2 - c3, c3 - c4, c4])
    total = float(logits.size)
    w_b = jnp.where(c_b > 0, total / ((1.0 - 0.5) * c_b), 0.0)
    return jnp.sum(w_b * s_b) / targets.shape[0]


# speedup vs baseline: 1.1843x; 1.0660x over previous
import jax
import jax.numpy as jnp
from jax import lax
from jax.experimental import pallas as pl
from jax.experimental.pallas import tpu as pltpu
from jax.experimental.pallas import tpu_sc as plsc

N = 4_000_000
BINS = 5
CHUNK = 16_000
NCHUNKS = N // CHUNK
TCC = 160
SCC = NCHUNKS - TCC
TC_RB = 200
TC_STEPS = TCC * 125 // TC_RB
NC, NS = 2, 16
NW = NC * NS
VPC = CHUNK // 16

_BASE = SCC // NW
_EXTRA = SCC % NW

_LT = (
    0.2876902085936771, -1.333342676597351, 0.8865566226237033,
    -0.7874456068595445, 0.8869645527936711, -0.9538804877894336,
)


BPC = CHUNK // 128


def _sc_body(x_hbm, y_hbm, out_hbm,
             xb0, yb0, xb1, yb1, acc_s, acc_c, sx0, sy0, sx1, sy1):
    cid = lax.axis_index("c")
    sid = lax.axis_index("s")
    wid = sid * NC + cid
    nch = jnp.where(wid < _EXTRA, _BASE + 1, _BASE)
    start = TCC + wid * _BASE + jnp.minimum(wid, _EXTRA)

    zeros16 = jnp.zeros((16,), jnp.float32)
    for k in range(BINS):
        acc_s[pl.ds(k * 16, 16)] = zeros16
        acc_c[pl.ds(k * 16, 16)] = zeros16

    iota = lax.iota(jnp.int32, 16)
    ones16 = jnp.ones((16,), jnp.float32)

    def start_dma(c, xb, yb, sx, sy):
        pltpu.async_copy(x_hbm.at[pl.ds(c * BPC, BPC), 1, :], xb, sx)
        pltpu.async_copy(y_hbm.at[pl.ds(c * CHUNK, CHUNK)], yb, sy)

    def wait_dma(c, xb, yb, sx, sy):
        pltpu.make_async_copy(x_hbm.at[pl.ds(c * BPC, BPC), 1, :], xb, sx).wait()
        pltpu.make_async_copy(y_hbm.at[pl.ds(c * CHUNK, CHUNK)], yb, sy).wait()

    def process(xb, yb):
        @plsc.parallel_loop(0, BPC, unroll=1)
        def _(blk):
            for k in range(8):
                y = yb[pl.ds(blk * 128 + k * 16, 16)]
                x = xb[blk, pl.ds(k * 16, 16)]
                e = jnp.exp(-x)
                r = 1.0 / (1.0 + e)
                g = jnp.abs(r - y)
                u = jnp.maximum(r, 1.0 - r) - 0.75
                p = jnp.float32(_LT[5])
                for j in (4, 3, 2, 1, 0):
                    p = p * u + _LT[j]
                pe = jnp.maximum(x, 0.0) - x * y + p
                b = jnp.minimum((g * 5.0).astype(jnp.int32), 4)
                idx = b * 16 + iota
                plsc.addupdate_scatter(acc_s, [idx], pe)
                plsc.addupdate_scatter(acc_c, [idx], ones16)

    start_dma(start, xb0, yb0, sx0, sy0)

    def chunk_body(j, carry):
        c = start + j
        even = (j % 2) == 0

        @pl.when(even)
        def _():
            @pl.when(j + 1 < nch)
            def _():
                start_dma(c + 1, xb1, yb1, sx1, sy1)
            wait_dma(c, xb0, yb0, sx0, sy0)
            process(xb0, yb0)

        @pl.when(jnp.logical_not(even))
        def _():
            @pl.when(j + 1 < nch)
            def _():
                start_dma(c + 1, xb0, yb0, sx0, sy0)
            wait_dma(c, xb1, yb1, sx1, sy1)
            process(xb1, yb1)

        return carry

    lax.fori_loop(0, nch, chunk_body, 0)
    pltpu.sync_copy(acc_s, out_hbm.at[pl.ds(wid * 160, 80)])
    pltpu.sync_copy(acc_c, out_hbm.at[pl.ds(wid * 160 + 80, 80)])


@jax.jit
def _ghm_sc(x, targets):
    mesh = plsc.VectorSubcoreMesh(
        core_axis_name="c", subcore_axis_name="s", num_cores=NC, num_subcores=NS
    )
    run = pl.kernel(
        _sc_body,
        out_type=jax.ShapeDtypeStruct((NW * 160,), jnp.float32),
        mesh=mesh,
        scratch_types=[
            pltpu.VMEM((BPC, 128), jnp.float32),
            pltpu.VMEM((CHUNK,), jnp.float32),
            pltpu.VMEM((BPC, 128), jnp.float32),
            pltpu.VMEM((CHUNK,), jnp.float32),
            pltpu.VMEM((80,), jnp.float32),
            pltpu.VMEM((80,), jnp.float32),
            pltpu.SemaphoreType.DMA,
            pltpu.SemaphoreType.DMA,
            pltpu.SemaphoreType.DMA,
            pltpu.SemaphoreType.DMA,
        ],
        compiler_params=pltpu.CompilerParams(needs_layout_passes=False),
    )
    return run(x, targets)


RTC = TCC * 125
TC_PH = 4
HR = RTC // TC_PH


def _tc_body(x3_ref, y_ref, out_ref, xb, yb, sx, sy):
    def start(h):
        s = h % 2
        pltpu.make_async_copy(
            x3_ref.at[pl.ds(h * HR, HR), 1, :], xb.at[s], sx.at[s]).start()
        pltpu.make_async_copy(
            y_ref.at[pl.ds(h * HR, HR), :], yb.at[s], sy.at[s]).start()

    def wait(h):
        s = h % 2
        pltpu.make_async_copy(
            x3_ref.at[pl.ds(h * HR, HR), 1, :], xb.at[s], sx.at[s]).wait()
        pltpu.make_async_copy(
            y_ref.at[pl.ds(h * HR, HR), :], yb.at[s], sy.at[s]).wait()

    def strip_fn(xh, yh):
        def strip(s, acc):
            x = xh[pl.ds(8 * s, 8), :]
            y = yh[pl.ds(8 * s, 8), :]
            e = jnp.exp(-x)
            r = 1.0 / (1.0 + e)
            g = jnp.abs(r - y)
            pe = jnp.maximum(x, 0.0) - x * y - jnp.log(jnp.maximum(r, 1.0 - r))
            new = [acc[0] + pe]
            for k in (1, 2, 3, 4):
                m = g >= (k / 5.0)
                new.append(acc[k] + jnp.where(m, pe, 0.0))
            for k in (1, 2, 3, 4):
                m = g >= (k / 5.0)
                new.append(acc[4 + k] + jnp.where(m, 1.0, 0.0))
            return tuple(new)
        return strip

    z8 = jnp.zeros((8, 128), jnp.float32)
    acc = (z8,) * 9
    start(0)
    for h in range(TC_PH):
        if h + 1 < TC_PH:
            start(h + 1)
        wait(h)
        s = h % 2
        acc = lax.fori_loop(0, HR // 8, strip_fn(xb.at[s], yb.at[s]), acc,
                            unroll=4)
    for k in range(9):
        out_ref[8 * k:8 * k + 8, :] = acc[k]


@jax.jit
def _ghm_tc(x3, y2):
    return pl.pallas_call(
        _tc_body,
        in_specs=[
            pl.BlockSpec(memory_space=pl.ANY),
            pl.BlockSpec(memory_space=pl.ANY),
        ],
        out_specs=pl.BlockSpec(memory_space=pltpu.MemorySpace.VMEM),
        out_shape=jax.ShapeDtypeStruct((72, 128), jnp.float32),
        scratch_shapes=[
            pltpu.VMEM((2, HR, 128), jnp.float32),
            pltpu.VMEM((2, HR, 128), jnp.float32),
            pltpu.SemaphoreType.DMA((2,)),
            pltpu.SemaphoreType.DMA((2,)),
        ],
    )(x3, y2)


def kernel(logits, targets):
    x3 = logits.reshape(N // 128, 128, 2).transpose(0, 2, 1)
    part = _ghm_sc(x3, targets)
    y2 = targets.reshape(N // 128, 128)
    tcp = _ghm_tc(x3, y2)
    p = part.reshape(NW, 2, BINS, 16)
    s_b = jnp.sum(p[:, 0], axis=(0, 2))
    c_b = jnp.sum(p[:, 1], axis=(0, 2))
    q = jnp.sum(tcp.reshape(9, 8, 128), axis=(1, 2))
    dif = jnp.array(
        [[1, -1, 0, 0, 0], [0, 1, -1, 0, 0], [0, 0, 1, -1, 0],
         [0, 0, 0, 1, -1], [0, 0, 0, 0, 1]], dtype=jnp.float32)
    ntc = float(TCC * CHUNK)
    qs = q[0:5]
    qc = jnp.concatenate([jnp.full((1,), ntc, jnp.float32), q[5:9]])
    s_b = s_b + dif @ qs
    c_b = c_b + dif @ qc
    total = float(logits.size)
    w_b = jnp.where(c_b > 0, total / ((1.0 - 0.5) * c_b), 0.0)
    return jnp.sum(w_b * s_b) / targets.shape[0]


# speedup vs baseline: 1.1928x; 1.0072x over previous
import jax
import jax.numpy as jnp
from jax import lax
from jax.experimental import pallas as pl
from jax.experimental.pallas import tpu as pltpu
from jax.experimental.pallas import tpu_sc as plsc

N = 4_000_000
BINS = 5
CHUNK = 16_000
NCHUNKS = N // CHUNK
TCC = 160
SCC = NCHUNKS - TCC
TC_RB = 200
TC_STEPS = TCC * 125 // TC_RB
NC, NS = 2, 16
NW = NC * NS
VPC = CHUNK // 16

_BASE = SCC // NW
_EXTRA = SCC % NW

_LT = (
    0.2876902085936771, -1.333342676597351, 0.8865566226237033,
    -0.7874456068595445, 0.8869645527936711, -0.9538804877894336,
)


BPC = CHUNK // 128


def _sc_body(x_hbm, y_hbm, out_hbm,
             xb0, yb0, xb1, yb1, acc_s, acc_c, sx0, sy0, sx1, sy1):
    cid = lax.axis_index("c")
    sid = lax.axis_index("s")
    wid = sid * NC + cid
    nch = jnp.where(wid < _EXTRA, _BASE + 1, _BASE)
    start = TCC + wid * _BASE + jnp.minimum(wid, _EXTRA)

    zeros16 = jnp.zeros((16,), jnp.float32)
    for k in range(BINS):
        acc_s[pl.ds(k * 16, 16)] = zeros16
        acc_c[pl.ds(k * 16, 16)] = zeros16

    iota = lax.iota(jnp.int32, 16)
    ones16 = jnp.ones((16,), jnp.float32)

    def start_dma(c, xb, yb, sx, sy):
        pltpu.async_copy(x_hbm.at[pl.ds(c * BPC, BPC), 1, :], xb, sx)
        pltpu.async_copy(y_hbm.at[pl.ds(c * CHUNK, CHUNK)], yb, sy)

    def wait_dma(c, xb, yb, sx, sy):
        pltpu.make_async_copy(x_hbm.at[pl.ds(c * BPC, BPC), 1, :], xb, sx).wait()
        pltpu.make_async_copy(y_hbm.at[pl.ds(c * CHUNK, CHUNK)], yb, sy).wait()

    def process(xb, yb):
        @plsc.parallel_loop(0, BPC, unroll=1)
        def _(blk):
            for k in range(8):
                y = yb[pl.ds(blk * 128 + k * 16, 16)]
                x = xb[blk, pl.ds(k * 16, 16)]
                e = jnp.exp(-x)
                r = 1.0 / (1.0 + e)
                g = jnp.abs(r - y)
                u = jnp.maximum(r, 1.0 - r) - 0.75
                p = jnp.float32(_LT[5])
                for j in (4, 3, 2, 1, 0):
                    p = p * u + _LT[j]
                pe = jnp.maximum(x, 0.0) - x * y + p
                b = jnp.minimum((g * 5.0).astype(jnp.int32), 4)
                idx = b * 16 + iota
                plsc.addupdate_scatter(acc_s, [idx], pe)
                plsc.addupdate_scatter(acc_c, [idx], ones16)

    start_dma(start, xb0, yb0, sx0, sy0)

    def chunk_body(j, carry):
        c = start + j
        even = (j % 2) == 0

        @pl.when(even)
        def _():
            @pl.when(j + 1 < nch)
            def _():
                start_dma(c + 1, xb1, yb1, sx1, sy1)
            wait_dma(c, xb0, yb0, sx0, sy0)
            process(xb0, yb0)

        @pl.when(jnp.logical_not(even))
        def _():
            @pl.when(j + 1 < nch)
            def _():
                start_dma(c + 1, xb0, yb0, sx0, sy0)
            wait_dma(c, xb1, yb1, sx1, sy1)
            process(xb1, yb1)

        return carry

    lax.fori_loop(0, nch, chunk_body, 0)
    pltpu.sync_copy(acc_s, out_hbm.at[pl.ds(wid * 160, 80)])
    pltpu.sync_copy(acc_c, out_hbm.at[pl.ds(wid * 160 + 80, 80)])


@jax.jit
def _ghm_sc(x, targets):
    mesh = plsc.VectorSubcoreMesh(
        core_axis_name="c", subcore_axis_name="s", num_cores=NC, num_subcores=NS
    )
    run = pl.kernel(
        _sc_body,
        out_type=jax.ShapeDtypeStruct((NW * 160,), jnp.float32),
        mesh=mesh,
        scratch_types=[
            pltpu.VMEM((BPC, 128), jnp.float32),
            pltpu.VMEM((CHUNK,), jnp.float32),
            pltpu.VMEM((BPC, 128), jnp.float32),
            pltpu.VMEM((CHUNK,), jnp.float32),
            pltpu.VMEM((80,), jnp.float32),
            pltpu.VMEM((80,), jnp.float32),
            pltpu.SemaphoreType.DMA,
            pltpu.SemaphoreType.DMA,
            pltpu.SemaphoreType.DMA,
            pltpu.SemaphoreType.DMA,
        ],
        compiler_params=pltpu.CompilerParams(needs_layout_passes=False),
    )
    return run(x, targets)


RTC = TCC * 125
TC_PH = 4
HR = RTC // TC_PH


def _tc_body(x3_ref, y_ref, out_ref, xb, yb, sx, sy):
    def start(h):
        s = h % 2
        pltpu.make_async_copy(
            x3_ref.at[pl.ds(h * HR, HR), 1, :], xb.at[s], sx.at[s]).start()
        pltpu.make_async_copy(
            y_ref.at[pl.ds(h * HR, HR), :], yb.at[s], sy.at[s]).start()

    def wait(h):
        s = h % 2
        pltpu.make_async_copy(
            x3_ref.at[pl.ds(h * HR, HR), 1, :], xb.at[s], sx.at[s]).wait()
        pltpu.make_async_copy(
            y_ref.at[pl.ds(h * HR, HR), :], yb.at[s], sy.at[s]).wait()

    def strip_fn(xh, yh):
        def strip(s, acc):
            x = xh[pl.ds(8 * s, 8), :]
            y = yh[pl.ds(8 * s, 8), :]
            e = jnp.exp(-x)
            r = 1.0 / (1.0 + e)
            g = jnp.abs(r - y)
            pe = jnp.maximum(x, 0.0) - x * y - jnp.log(jnp.maximum(r, 1.0 - r))
            new = [acc[0] + pe]
            for k in (1, 2, 3, 4):
                m = g >= (k / 5.0)
                new.append(acc[k] + jnp.where(m, pe, 0.0))
            for k in (1, 2, 3, 4):
                m = g >= (k / 5.0)
                new.append(acc[4 + k] + jnp.where(m, 1.0, 0.0))
            return tuple(new)
        return strip

    z8 = jnp.zeros((8, 128), jnp.float32)
    acc = (z8,) * 9
    start(0)
    for h in range(TC_PH):
        if h + 1 < TC_PH:
            start(h + 1)
        wait(h)
        s = h % 2
        acc = lax.fori_loop(0, HR // 8, strip_fn(xb.at[s], yb.at[s]), acc,
                            unroll=4)
    for k in range(9):
        out_ref[8 * k:8 * k + 8, :] = acc[k]


@jax.jit
def _ghm_tc(x3, y2):
    return pl.pallas_call(
        _tc_body,
        in_specs=[
            pl.BlockSpec(memory_space=pl.ANY),
            pl.BlockSpec(memory_space=pl.ANY),
        ],
        out_specs=pl.BlockSpec(memory_space=pltpu.MemorySpace.VMEM),
        out_shape=jax.ShapeDtypeStruct((72, 128), jnp.float32),
        scratch_shapes=[
            pltpu.VMEM((2, HR, 128), jnp.float32),
            pltpu.VMEM((2, HR, 128), jnp.float32),
            pltpu.SemaphoreType.DMA((2,)),
            pltpu.SemaphoreType.DMA((2,)),
        ],
    )(x3, y2)


def kernel(logits, targets):
    x3 = logits.reshape(N // 128, 128, 2).transpose(0, 2, 1)
    part = _ghm_sc(x3, targets)
    y2 = targets.reshape(N // 128, 128)
    tcp = _ghm_tc(x3, y2)
    p = part.reshape(NW, 2, BINS, 16)
    s_b = jnp.sum(p[:, 0], axis=(0, 2))
    c_b = jnp.sum(p[:, 1], axis=(0, 2))
    q = jnp.sum(tcp.reshape(9, 8, 128), axis=(1, 2))
    ntc = float(TCC * CHUNK)
    qs = q[0:5]
    qc = jnp.concatenate([jnp.full((1,), ntc, jnp.float32), q[5:9]])
    z1 = jnp.zeros((1,), jnp.float32)
    s_b = s_b + qs - jnp.concatenate([qs[1:], z1])
    c_b = c_b + qc - jnp.concatenate([qc[1:], z1])
    total = float(logits.size)
    w_b = jnp.where(c_b > 0, total / ((1.0 - 0.5) * c_b), 0.0)
    return jnp.sum(w_b * s_b) / targets.shape[0]
